# baseline (speedup 1.0000x reference)
import functools

import jax
import jax.numpy as jnp
from jax import lax
from jax.experimental import pallas as pl
from jax.experimental.pallas import tpu as pltpu
from jax.experimental.pallas import tpu_sc as plsc

E = 8
D = 1024
F = 4096
T = 2048
BM = 128
S = T * 2 + E * BM
MB = S // BM
BF = 512
NF = F // BF
TCHUNK = 128
NCHUNK = T // TCHUNK

NW = 32
TPW = T // NW
CH = 16
NCH = TPW // CH


def _sc_mesh():
    return plsc.VectorSubcoreMesh(core_axis_name="c", subcore_axis_name="s")


def _gate_body(x_ref, wg_ref, e_ref, w_ref, r_ref, cnt_ref, carry_ref):
    c = pl.program_id(0)

    @pl.when(c == 0)
    def _init():
        carry_ref[...] = jnp.zeros_like(carry_ref)

    x = x_ref[...]
    wg = wg_ref[...]
    logits = jax.lax.dot_general(
        x, wg, (((1,), (1,)), ((), ())),
        preferred_element_type=jnp.float32)
    m = jnp.max(logits, axis=-1, keepdims=True)
    p = jnp.exp(logits - m)
    probs = p / jnp.sum(p, axis=-1, keepdims=True)

    idx = jax.lax.broadcasted_iota(jnp.int32, (TCHUNK, E), 1)
    m0 = jnp.max(logits, axis=-1, keepdims=True)
    e0 = jnp.min(jnp.where(logits >= m0, idx, E), axis=-1, keepdims=True)
    oh0 = (idx == e0)
    masked = jnp.where(oh0, -jnp.inf, logits)
    m1 = jnp.max(masked, axis=-1, keepdims=True)
    e1 = jnp.min(jnp.where(masked >= m1, idx, E), axis=-1, keepdims=True)
    oh1 = (idx == e1)

    oh0f = oh0.astype(jnp.float32)
    oh1f = oh1.astype(jnp.float32)
    p0 = jnp.sum(probs * oh0f, axis=-1, keepdims=True)
    p1 = jnp.sum(probs * oh1f, axis=-1, keepdims=True)
    tot = p0 + p1
    w0 = p0 / tot
    w1 = p1 / tot

    ii = jax.lax.broadcasted_iota(jnp.int32, (TCHUNK, TCHUNK), 0)
    jj = jax.lax.broadcasted_iota(jnp.int32, (TCHUNK, TCHUNK), 1)
    tril = (jj < ii).astype(jnp.float32)
    carry = carry_ref[...]
    r0 = jax.lax.dot_general(tril, oh0f, (((1,), (0,)), ((), ())),
                             preferred_element_type=jnp.float32)
    rank0 = jnp.sum((carry + r0) * oh0f, axis=-1, keepdims=True)
    mid = carry + jnp.sum(oh0f, axis=0, keepdims=True)
    r1 = jax.lax.dot_general(tril, oh1f, (((1,), (0,)), ((), ())),
                             preferred_element_type=jnp.float32)
    rank1 = jnp.sum((mid + r1) * oh1f, axis=-1, keepdims=True)
    new_carry = mid + jnp.sum(oh1f, axis=0, keepdims=True)
    carry_ref[...] = new_carry

    e_ref[...] = jnp.concatenate([e0, e1], axis=1)
    w_ref[...] = jnp.concatenate([w0, w1], axis=1)
    r_ref[...] = jnp.concatenate([rank0, rank1], axis=1)
    cnt_ref[...] = jnp.broadcast_to(new_carry, (8, E))


def _gate_call(x, wg, *, interpret=False):
    return pl.pallas_call(
        _gate_body,
        grid=(NCHUNK,),
        in_specs=[
            pl.BlockSpec((TCHUNK, D), lambda c: (c, 0)),
            pl.BlockSpec((E, D), lambda c: (0, 0)),
        ],
        out_specs=[
            pl.BlockSpec((TCHUNK, 2), lambda c: (c, 0)),
            pl.BlockSpec((TCHUNK, 2), lambda c: (c, 0)),
            pl.BlockSpec((TCHUNK, 2), lambda c: (c, 0)),
            pl.BlockSpec((8, E), lambda c: (0, 0)),
        ],
        out_shape=[
            jax.ShapeDtypeStruct((T, 2), jnp.int32),
            jax.ShapeDtypeStruct((T, 2), jnp.float32),
            jax.ShapeDtypeStruct((T, 2), jnp.float32),
            jax.ShapeDtypeStruct((8, E), jnp.float32),
        ],
        scratch_shapes=[pltpu.VMEM((1, E), jnp.float32)],
        interpret=interpret,
    )(x, wg)


def _sc_dispatch(x, slots3d):
    @functools.partial(
        pl.kernel, mesh=_sc_mesh(),
        out_type=jax.ShapeDtypeStruct((S, D), jnp.float32),
        scratch_types=[
            pltpu.VMEM((2 * NCH, CH), jnp.int32),
            pltpu.VMEM((CH, D), jnp.float32),
            pltpu.SemaphoreType.DMA,
        ],
    )
    def k(x_hbm, slots_hbm, xs_hbm, idx_v, rows_v, sem):
        wid = lax.axis_index("s") * 2 + lax.axis_index("c")
        base = wid * TPW
        pltpu.sync_copy(slots_hbm.at[wid], idx_v)
        for c in range(NCH):
            pltpu.sync_copy(x_hbm.at[pl.ds(base + c * CH, CH)], rows_v)
            ca = pltpu.async_copy(rows_v, xs_hbm.at[idx_v.at[2 * c]], sem)
            cb = pltpu.async_copy(rows_v, xs_hbm.at[idx_v.at[2 * c + 1]], sem)
            ca.wait()
            cb.wait()

    return k(x, slots3d)


def _sc_combine(y, slots3d, wb0, wb1):
    @functools.partial(
        pl.kernel, mesh=_sc_mesh(),
        out_type=jax.ShapeDtypeStruct((T, D), jnp.float32),
        scratch_types=[
            pltpu.VMEM((2 * NCH, CH), jnp.int32),
            pltpu.VMEM((CH, D), jnp.float32),
            pltpu.VMEM((CH, D), jnp.float32),
            pltpu.VMEM((TPW, 16), jnp.float32),
            pltpu.VMEM((TPW, 16), jnp.float32),
            pltpu.SemaphoreType.DMA,
            pltpu.SemaphoreType.DMA,
        ],
    )
    def k(y_hbm, slots_hbm, wb0_hbm, wb1_hbm, out_hbm,
          idx_v, b0, b1, w0v, w1v, sem0, sem1):
        wid = lax.axis_index("s") * 2 + lax.axis_index("c")
        base = wid * TPW
        pltpu.sync_copy(slots_hbm.at[wid], idx_v)
        pltpu.sync_copy(wb0_hbm.at[pl.ds(base, TPW)], w0v)
        pltpu.sync_copy(wb1_hbm.at[pl.ds(base, TPW)], w1v)
        for c in range(NCH):
            ca = pltpu.async_copy(y_hbm.at[idx_v.at[2 * c]], b0, sem0)
            cb = pltpu.async_copy(y_hbm.at[idx_v.at[2 * c + 1]], b1, sem1)
            ca.wait()
            cb.wait()

            def token_body(j, _):
                w0 = w0v[c * CH + j, :]
                w1 = w1v[c * CH + j, :]
                for l in range(D // 16):
                    b0[j, pl.ds(l * 16, 16)] = (
                        w0 * b0[j, pl.ds(l * 16, 16)]
                        + w1 * b1[j, pl.ds(l * 16, 16)])
                return 0

            jax.lax.fori_loop(0, CH, token_body, 0)
            pltpu.sync_copy(b0, out_hbm.at[pl.ds(base + c * CH, CH)])

    return k(y, slots3d, wb0, wb1)


def _ffn_body(meta_ref, w1_ref, w3_ref, w2_ref, xs_hbm, y_hbm,
              xs_v, y_v, sem_in, sem_out):
    e = pl.program_id(0)
    f = pl.program_id(1)

    @pl.when((e == 0) & (f == 0))
    def _load_xs():
        cp = pltpu.make_async_copy(xs_hbm, xs_v, sem_in)
        cp.start()
        y_v[...] = jnp.zeros_like(y_v)
        cp.wait()

    nb = meta_ref[e]
    base = meta_ref[E + e]

    def blk(b, carry):
        r0 = (base + b) * BM
        x = xs_v[pl.ds(r0, BM), :]
        a = jax.lax.dot_general(x, w1_ref[0], (((1,), (1,)), ((), ())),
                                preferred_element_type=jnp.float32)
        g = jax.lax.dot_general(x, w3_ref[0], (((1,), (1,)), ((), ())),
                                preferred_element_type=jnp.float32)
        h = (a * jax.lax.logistic(a)) * g
        yp = jax.lax.dot_general(h, w2_ref[0], (((1,), (1,)), ((), ())),
                                 preferred_element_type=jnp.float32)
        prev = y_v[pl.ds(r0, BM), :]
        y_v[pl.ds(r0, BM), :] = jnp.where(f == 0, yp, prev + yp)
        return carry

    jax.lax.fori_loop(0, nb, blk, 0)

    @pl.when((e == E - 1) & (f == NF - 1))
    def _store_y():
        cp = pltpu.make_async_copy(y_v, y_hbm, sem_out)
        cp.start()
        cp.wait()


def _ffn_call(meta, xs, W1, W3, W2, *, interpret=False):
    grid_spec = pltpu.PrefetchScalarGridSpec(
        num_scalar_prefetch=1,
        grid=(E, NF),
        in_specs=[
            pl.BlockSpec((1, BF, D), lambda e, f, meta: (e, f, 0)),
            pl.BlockSpec((1, BF, D), lambda e, f, meta: (e, f, 0)),
            pl.BlockSpec((1, D, BF), lambda e, f, meta: (e, 0, f)),
            pl.BlockSpec(memory_space=pl.ANY),
        ],
        out_specs=pl.BlockSpec(memory_space=pl.ANY),
        scratch_shapes=[
            pltpu.VMEM((S, D), jnp.float32),
            pltpu.VMEM((S, D), jnp.float32),
            pltpu.SemaphoreType.DMA,
            pltpu.SemaphoreType.DMA,
        ],
    )
    return pl.pallas_call(
        _ffn_body,
        grid_spec=grid_spec,
        out_shape=jax.ShapeDtypeStruct((S, D), jnp.float32),
        interpret=interpret,
    )(meta, W1, W3, W2, xs)


def _moe_impl(hidden_states, Wg, W1, W2, W3, *, interpret=False):
    x = hidden_states.reshape(T, D)
    e, w, r, cnt = _gate_call(x, Wg, interpret=interpret)
    counts = cnt[0].astype(jnp.int32)
    pblocks = (counts + BM - 1) // BM
    starts = jnp.concatenate(
        [jnp.zeros((1,), jnp.int32), jnp.cumsum(pblocks)[:-1]])
    off = starts * BM
    slot = jnp.take(off, e, axis=0) + r.astype(jnp.int32)
    meta = jnp.concatenate([pblocks, starts]).astype(jnp.int32)
    slots3d = slot.reshape(NW, NCH, CH, 2).transpose(0, 1, 3, 2).reshape(
        NW, 2 * NCH, CH)
    wb0 = jnp.broadcast_to(w[:, 0:1], (T, 16))
    wb1 = jnp.broadcast_to(w[:, 1:2], (T, 16))
    xs = _sc_dispatch(x, slots3d)
    y = _ffn_call(meta, xs, W1, W3, W2, interpret=interpret)
    out = _sc_combine(y, slots3d, wb0, wb1)
    return out.reshape(hidden_states.shape)


def kernel(hidden_states, Wg, W1, W2, W3):
    return _moe_impl(hidden_states, Wg, W1, W2, W3)


# speedup vs baseline: 1.0287x; 1.0287x over previous
import functools

import jax
import jax.numpy as jnp
from jax import lax
from jax.experimental import pallas as pl
from jax.experimental.pallas import tpu as pltpu
from jax.experimental.pallas import tpu_sc as plsc

E = 8
D = 1024
F = 4096
T = 2048
BM = 128
S = T * 2 + E * BM
MB = S // BM
BF = 512
NF = F // BF
TCHUNK = 128
NCHUNK = T // TCHUNK

NW = 32
TPW = T // NW
CH = 16
NCH = TPW // CH


def _sc_mesh():
    return plsc.VectorSubcoreMesh(core_axis_name="c", subcore_axis_name="s")


def _gate_body(x_ref, wg_ref, e_ref, w_ref, r_ref, cnt_ref, carry_ref):
    c = pl.program_id(0)

    @pl.when(c == 0)
    def _init():
        carry_ref[...] = jnp.zeros_like(carry_ref)

    x = x_ref[...]
    wg = wg_ref[...]
    logits = jax.lax.dot_general(
        x, wg, (((1,), (1,)), ((), ())),
        preferred_element_type=jnp.float32)
    m = jnp.max(logits, axis=-1, keepdims=True)
    p = jnp.exp(logits - m)
    probs = p / jnp.sum(p, axis=-1, keepdims=True)

    idx = jax.lax.broadcasted_iota(jnp.int32, (TCHUNK, E), 1)
    m0 = jnp.max(logits, axis=-1, keepdims=True)
    e0 = jnp.min(jnp.where(logits >= m0, idx, E), axis=-1, keepdims=True)
    oh0 = (idx == e0)
    masked = jnp.where(oh0, -jnp.inf, logits)
    m1 = jnp.max(masked, axis=-1, keepdims=True)
    e1 = jnp.min(jnp.where(masked >= m1, idx, E), axis=-1, keepdims=True)
    oh1 = (idx == e1)

    oh0f = oh0.astype(jnp.float32)
    oh1f = oh1.astype(jnp.float32)
    p0 = jnp.sum(probs * oh0f, axis=-1, keepdims=True)
    p1 = jnp.sum(probs * oh1f, axis=-1, keepdims=True)
    tot = p0 + p1
    w0 = p0 / tot
    w1 = p1 / tot

    ii = jax.lax.broadcasted_iota(jnp.int32, (TCHUNK, TCHUNK), 0)
    jj = jax.lax.broadcasted_iota(jnp.int32, (TCHUNK, TCHUNK), 1)
    tril = (jj < ii).astype(jnp.float32)
    carry = carry_ref[...]
    r0 = jax.lax.dot_general(tril, oh0f, (((1,), (0,)), ((), ())),
                             preferred_element_type=jnp.float32)
    rank0 = jnp.sum((carry + r0) * oh0f, axis=-1, keepdims=True)
    mid = carry + jnp.sum(oh0f, axis=0, keepdims=True)
    r1 = jax.lax.dot_general(tril, oh1f, (((1,), (0,)), ((), ())),
                             preferred_element_type=jnp.float32)
    rank1 = jnp.sum((mid + r1) * oh1f, axis=-1, keepdims=True)
    new_carry = mid + jnp.sum(oh1f, axis=0, keepdims=True)
    carry_ref[...] = new_carry

    e_ref[...] = jnp.concatenate([e0, e1], axis=1)
    w_ref[...] = jnp.concatenate([w0, w1], axis=1)
    r_ref[...] = jnp.concatenate([rank0, rank1], axis=1)
    cnt_ref[...] = jnp.broadcast_to(new_carry, (8, E))


def _gate_call(x, wg, *, interpret=False):
    return pl.pallas_call(
        _gate_body,
        grid=(NCHUNK,),
        in_specs=[
            pl.BlockSpec((TCHUNK, D), lambda c: (c, 0)),
            pl.BlockSpec((E, D), lambda c: (0, 0)),
        ],
        out_specs=[
            pl.BlockSpec((TCHUNK, 2), lambda c: (c, 0)),
            pl.BlockSpec((TCHUNK, 2), lambda c: (c, 0)),
            pl.BlockSpec((TCHUNK, 2), lambda c: (c, 0)),
            pl.BlockSpec((8, E), lambda c: (0, 0)),
        ],
        out_shape=[
            jax.ShapeDtypeStruct((T, 2), jnp.int32),
            jax.ShapeDtypeStruct((T, 2), jnp.float32),
            jax.ShapeDtypeStruct((T, 2), jnp.float32),
            jax.ShapeDtypeStruct((8, E), jnp.float32),
        ],
        scratch_shapes=[pltpu.VMEM((1, E), jnp.float32)],
        interpret=interpret,
    )(x, wg)


def _sc_dispatch(x, slots3d):
    @functools.partial(
        pl.kernel, mesh=_sc_mesh(),
        out_type=jax.ShapeDtypeStruct((S, D), jnp.float32),
        scratch_types=[
            pltpu.VMEM((2 * NCH, CH), jnp.int32),
            pltpu.VMEM((CH, D), jnp.float32),
            pltpu.VMEM((CH, D), jnp.float32),
            pltpu.SemaphoreType.DMA,
            pltpu.SemaphoreType.DMA,
            pltpu.SemaphoreType.DMA,
        ],
    )
    def k(x_hbm, slots_hbm, xs_hbm, idx_v, rows_a, rows_b, lsem_a, lsem_b, ssem):
        wid = lax.axis_index("s") * 2 + lax.axis_index("c")
        base = wid * TPW
        pltpu.sync_copy(slots_hbm.at[wid], idx_v)
        bufs = (rows_a, rows_b)
        lsems = (lsem_a, lsem_b)

        def load(c):
            return pltpu.async_copy(
                x_hbm.at[pl.ds(base + c * CH, CH)], bufs[c % 2], lsems[c % 2])

        loads = [load(0), load(1)]
        for c in range(NCH):
            loads[c].wait()
            ca = pltpu.async_copy(bufs[c % 2], xs_hbm.at[idx_v.at[2 * c]], ssem)
            cb = pltpu.async_copy(bufs[c % 2], xs_hbm.at[idx_v.at[2 * c + 1]],
                                  ssem)
            ca.wait()
            cb.wait()
            if c + 2 < NCH:
                loads.append(load(c + 2))

    return k(x, slots3d)


def _sc_combine(y, slots3d, wb0, wb1):
    @functools.partial(
        pl.kernel, mesh=_sc_mesh(),
        out_type=jax.ShapeDtypeStruct((T, D), jnp.float32),
        scratch_types=[
            pltpu.VMEM((2 * NCH, CH), jnp.int32),
            pltpu.VMEM((CH, D), jnp.float32),
            pltpu.VMEM((CH, D), jnp.float32),
            pltpu.VMEM((CH, D), jnp.float32),
            pltpu.VMEM((CH, D), jnp.float32),
            pltpu.VMEM((TPW, 16), jnp.float32),
            pltpu.VMEM((TPW, 16), jnp.float32),
            pltpu.SemaphoreType.DMA,
            pltpu.SemaphoreType.DMA,
            pltpu.SemaphoreType.DMA,
            pltpu.SemaphoreType.DMA,
            pltpu.SemaphoreType.DMA,
        ],
    )
    def k(y_hbm, slots_hbm, wb0_hbm, wb1_hbm, out_hbm,
          idx_v, b0a, b0b, b1a, b1b, w0v, w1v,
          g0a, g0b, g1a, g1b, stsem):
        wid = lax.axis_index("s") * 2 + lax.axis_index("c")
        base = wid * TPW
        pltpu.sync_copy(slots_hbm.at[wid], idx_v)
        pltpu.sync_copy(wb0_hbm.at[pl.ds(base, TPW)], w0v)
        pltpu.sync_copy(wb1_hbm.at[pl.ds(base, TPW)], w1v)
        bufs0 = (b0a, b0b)
        bufs1 = (b1a, b1b)
        sems0 = (g0a, g0b)
        sems1 = (g1a, g1b)

        def gather(c):
            p = c % 2
            return (
                pltpu.async_copy(y_hbm.at[idx_v.at[2 * c]], bufs0[p], sems0[p]),
                pltpu.async_copy(y_hbm.at[idx_v.at[2 * c + 1]], bufs1[p],
                                 sems1[p]),
            )

        gathers = [gather(0), gather(1)]
        for c in range(NCH):
            p = c % 2
            ga, gb = gathers[c]
            ga.wait()
            gb.wait()
            b0 = bufs0[p]
            b1 = bufs1[p]

            def token_body(j, _):
                w0 = w0v[c * CH + j, :]
                w1 = w1v[c * CH + j, :]
                for l in range(D // 16):
                    b0[j, pl.ds(l * 16, 16)] = (
                        w0 * b0[j, pl.ds(l * 16, 16)]
                        + w1 * b1[j, pl.ds(l * 16, 16)])
                return 0

            jax.lax.fori_loop(0, CH, token_body, 0)
            st = pltpu.async_copy(b0, out_hbm.at[pl.ds(base + c * CH, CH)],
                                  stsem)
            st.wait()
            if c + 2 < NCH:
                gathers.append(gather(c + 2))

    return k(y, slots3d, wb0, wb1)


def _ffn_body(meta_ref, w1_ref, w3_ref, w2_ref, xs_hbm, y_hbm,
              xs_v, y_v, sem_in, sem_out):
    e = pl.program_id(0)
    f = pl.program_id(1)

    @pl.when((e == 0) & (f == 0))
    def _load_xs():
        cp = pltpu.make_async_copy(xs_hbm, xs_v, sem_in)
        cp.start()
        y_v[...] = jnp.zeros_like(y_v)
        cp.wait()

    nb = meta_ref[e]
    base = meta_ref[E + e]

    def blk(b, carry):
        r0 = (base + b) * BM
        x = xs_v[pl.ds(r0, BM), :]
        a = jax.lax.dot_general(x, w1_ref[0], (((1,), (1,)), ((), ())),
                                preferred_element_type=jnp.float32)
        g = jax.lax.dot_general(x, w3_ref[0], (((1,), (1,)), ((), ())),
                                preferred_element_type=jnp.float32)
        h = (a * jax.lax.logistic(a)) * g
        yp = jax.lax.dot_general(h, w2_ref[0], (((1,), (1,)), ((), ())),
                                 preferred_element_type=jnp.float32)
        prev = y_v[pl.ds(r0, BM), :]
        y_v[pl.ds(r0, BM), :] = jnp.where(f == 0, yp, prev + yp)
        return carry

    jax.lax.fori_loop(0, nb, blk, 0)

    @pl.when((e == E - 1) & (f == NF - 1))
    def _store_y():
        cp = pltpu.make_async_copy(y_v, y_hbm, sem_out)
        cp.start()
        cp.wait()


def _ffn_call(meta, xs, W1, W3, W2, *, interpret=False):
    grid_spec = pltpu.PrefetchScalarGridSpec(
        num_scalar_prefetch=1,
        grid=(E, NF),
        in_specs=[
            pl.BlockSpec((1, BF, D), lambda e, f, meta: (e, f, 0)),
            pl.BlockSpec((1, BF, D), lambda e, f, meta: (e, f, 0)),
            pl.BlockSpec((1, D, BF), lambda e, f, meta: (e, 0, f)),
            pl.BlockSpec(memory_space=pl.ANY),
        ],
        out_specs=pl.BlockSpec(memory_space=pl.ANY),
        scratch_shapes=[
            pltpu.VMEM((S, D), jnp.float32),
            pltpu.VMEM((S, D), jnp.float32),
            pltpu.SemaphoreType.DMA,
            pltpu.SemaphoreType.DMA,
        ],
    )
    return pl.pallas_call(
        _ffn_body,
        grid_spec=grid_spec,
        out_shape=jax.ShapeDtypeStruct((S, D), jnp.float32),
        interpret=interpret,
    )(meta, W1, W3, W2, xs)


def _moe_impl(hidden_states, Wg, W1, W2, W3, *, interpret=False):
    x = hidden_states.reshape(T, D)
    e, w, r, cnt = _gate_call(x, Wg, interpret=interpret)
    counts = cnt[0].astype(jnp.int32)
    pblocks = (counts + BM - 1) // BM
    starts = jnp.concatenate(
        [jnp.zeros((1,), jnp.int32), jnp.cumsum(pblocks)[:-1]])
    off = starts * BM
    slot = jnp.take(off, e, axis=0) + r.astype(jnp.int32)
    meta = jnp.concatenate([pblocks, starts]).astype(jnp.int32)
    slots3d = slot.reshape(NW, NCH, CH, 2).transpose(0, 1, 3, 2).reshape(
        NW, 2 * NCH, CH)
    wb0 = jnp.broadcast_to(w[:, 0:1], (T, 16))
    wb1 = jnp.broadcast_to(w[:, 1:2], (T, 16))
    xs = _sc_dispatch(x, slots3d)
    y = _ffn_call(meta, xs, W1, W3, W2, interpret=interpret)
    out = _sc_combine(y, slots3d, wb0, wb1)
    return out.reshape(hidden_states.shape)


def kernel(hidden_states, Wg, W1, W2, W3):
    return _moe_impl(hidden_states, Wg, W1, W2, W3)
